# baseline (speedup 1.0000x reference)
import jax
import jax.numpy as jnp
from jax import lax
from jax.experimental import pallas as pl

_EXPERTS = 64
_BT = 4096


def _router_body(hs_ref, w_ref, b_ref, scores_ref, packed_ref):
    logits = lax.dot_general(
        hs_ref[...], w_ref[...], (((1,), (1,)), ((), ())),
        preferred_element_type=jnp.float32,
    )
    logits = logits + b_ref[...]
    ex = lax.broadcasted_iota(jnp.int32, logits.shape, 1)
    m1 = jnp.max(logits, axis=1, keepdims=True)
    i1 = jnp.min(jnp.where(logits == m1, ex, _EXPERTS), axis=1, keepdims=True)
    masked = jnp.where(ex == i1, -jnp.inf, logits)
    m2 = jnp.max(masked, axis=1, keepdims=True)
    i2 = jnp.min(jnp.where(masked == m2, ex, _EXPERTS), axis=1, keepdims=True)
    e = jnp.exp(m2 - m1)
    p1 = 1.0 / (1.0 + e)
    p2 = e / (1.0 + e)
    scores_ref[...] = jnp.where(ex == i1, p1, jnp.where(ex == i2, p2, 0.0))
    packed_ref[...] = jnp.reshape(i1 * _EXPERTS + i2, (logits.shape[0],))


def kernel(hidden_states, weight, bias):
    tokens, hidden = hidden_states.shape
    scores, packed = pl.pallas_call(
        _router_body,
        grid=(tokens // _BT,),
        in_specs=[
            pl.BlockSpec((_BT, hidden), lambda i: (i, 0)),
            pl.BlockSpec((_EXPERTS, hidden), lambda i: (0, 0)),
            pl.BlockSpec((1, _EXPERTS), lambda i: (0, 0)),
        ],
        out_specs=[
            pl.BlockSpec((_BT, _EXPERTS), lambda i: (i, 0)),
            pl.BlockSpec((_BT,), lambda i: (i,)),
        ],
        out_shape=[
            jax.ShapeDtypeStruct((tokens, _EXPERTS), jnp.float32),
            jax.ShapeDtypeStruct((tokens,), jnp.int32),
        ],
    )(hidden_states, weight, bias.reshape(1, _EXPERTS))
    idx = jnp.stack([packed // _EXPERTS, packed % _EXPERTS], axis=-1)
    return scores, idx
